# baseline (speedup 1.0000x reference)
import functools

import jax
import jax.numpy as jnp
import numpy as np
from jax import lax
from jax.experimental import pallas as pl
from jax.experimental.pallas import tpu as pltpu
from jax.experimental.pallas import tpu_sc as plsc

VOCAB = 100000
D_MODEL = 1024
MAX_LEN = 2048
EPS = 1e-5

NC, NS = 2, 16
NW = NC * NS

N_CHUNKS = 4
GATHER_CHUNK = 32
LN_ROWS = 256


def _pe_table(max_len, d_model):
    pos = np.arange(max_len, dtype=np.float32)[:, None]
    i = np.arange(0, d_model, 2, dtype=np.float32)
    div = np.exp(-np.log(10000.0) * i / d_model)
    pe = np.zeros((max_len, d_model), dtype=np.float32)
    pe[:, 0::2] = np.sin(pos * div)
    pe[:, 1::2] = np.cos(pos * div)
    return pe


_PE = _pe_table(MAX_LEN, D_MODEL)


def _sc_gather(table, idx):
    n = idx.shape[0]
    d = table.shape[1]
    b_per_w = n // NW
    chunk = min(GATHER_CHUNK, b_per_w)
    n_chunks = b_per_w // chunk
    mesh = plsc.VectorSubcoreMesh(core_axis_name="c", subcore_axis_name="s")

    @functools.partial(
        pl.kernel,
        out_type=jax.ShapeDtypeStruct((n, d), table.dtype),
        mesh=mesh,
        scratch_types=[
            pltpu.VMEM((b_per_w,), jnp.int32),
            pltpu.VMEM((chunk, d), table.dtype),
            pltpu.VMEM((chunk, d), table.dtype),
            pltpu.SemaphoreType.DMA,
            pltpu.SemaphoreType.DMA,
            pltpu.SemaphoreType.DMA,
            pltpu.SemaphoreType.DMA,
        ],
    )
    def gather_kernel(table_hbm, idx_hbm, out_hbm, idx_v, rows0, rows1,
                      gsem0, gsem1, wsem0, wsem1):
        wid = lax.axis_index("s") * NC + lax.axis_index("c")
        base = wid * b_per_w
        pltpu.sync_copy(idx_hbm.at[pl.ds(base, b_per_w)], idx_v)

        rows = (rows0, rows1)
        gsems = (gsem0, gsem1)
        wsems = (wsem0, wsem1)

        def start_gather(c, buf):
            return pltpu.async_copy(
                table_hbm.at[idx_v.at[pl.ds(c * chunk, chunk)]],
                rows[buf], gsems[buf])

        def start_write(c, buf):
            return pltpu.async_copy(
                rows[buf], out_hbm.at[pl.ds(base + c * chunk, chunk)],
                wsems[buf])

        start_gather(0, 0)
        writes = [None, None]
        for c in range(n_chunks):
            buf = c % 2
            pltpu.make_async_copy(
                table_hbm.at[idx_v.at[pl.ds(c * chunk, chunk)]],
                rows[buf], gsems[buf]).wait()
            if c + 1 < n_chunks:
                nbuf = (c + 1) % 2
                if writes[nbuf] is not None:
                    writes[nbuf].wait()
                    writes[nbuf] = None
                start_gather(c + 1, nbuf)
            writes[buf] = start_write(c, buf)
        for w in writes:
            if w is not None:
                w.wait()

    return gather_kernel(table, idx)


def _tc_ln_chunk(tok, pe, gamma, beta, out_prev, n_total, chunk_id):
    r, d = tok.shape
    l = pe.shape[0]
    blocks = r // LN_ROWS
    n_pe_blocks = l // LN_ROWS
    base = chunk_id * blocks

    def body(tok_ref, pe_ref, g_ref, b_ref, *rest):
        o_ref = rest[-1]
        i = pl.program_id(0)
        pe_blk = pe_ref[pl.ds((i % n_pe_blocks) * LN_ROWS, LN_ROWS), :]
        x = tok_ref[...] + pe_blk
        m = jnp.mean(x, axis=-1, keepdims=True)
        xc = x - m
        v = jnp.mean(xc * xc, axis=-1, keepdims=True)
        o_ref[...] = xc * lax.rsqrt(v + EPS) * g_ref[...] + b_ref[...]

    in_specs = [
        pl.BlockSpec((LN_ROWS, d), lambda i: (i, 0)),
        pl.BlockSpec((l, d), lambda i: (0, 0)),
        pl.BlockSpec((1, d), lambda i: (0, 0)),
        pl.BlockSpec((1, d), lambda i: (0, 0)),
    ]
    args = [tok, pe, gamma.reshape(1, d), beta.reshape(1, d)]
    aliases = {}
    if out_prev is not None:
        in_specs.append(pl.BlockSpec(memory_space=pl.ANY))
        args.append(out_prev)
        aliases = {4: 0}

    return pl.pallas_call(
        body,
        grid=(blocks,),
        in_specs=in_specs,
        out_specs=pl.BlockSpec((LN_ROWS, d), lambda i: (base + i, 0)),
        out_shape=jax.ShapeDtypeStruct((n_total, d), jnp.float32),
        input_output_aliases=aliases,
    )(*args)


def kernel(sequence, table, gamma, beta):
    b, l = sequence.shape
    d = table.shape[1]
    n = b * l
    idx = sequence.reshape(-1).astype(jnp.int32)
    pe = jnp.asarray(_PE[:l])

    rows_per_chunk = n // N_CHUNKS
    toks = [
        _sc_gather(table, lax.dynamic_slice_in_dim(idx, c * rows_per_chunk,
                                                   rows_per_chunk))
        for c in range(N_CHUNKS)
    ]
    out = None
    for c in range(N_CHUNKS):
        out = _tc_ln_chunk(toks[c], pe, gamma, beta, out, n, c)
    return out.reshape(b, l, d)


# speedup vs baseline: 1.1751x; 1.1751x over previous
import functools

import jax
import jax.numpy as jnp
import numpy as np
from jax import lax
from jax.experimental import pallas as pl
from jax.experimental.pallas import tpu as pltpu
from jax.experimental.pallas import tpu_sc as plsc

VOCAB = 100000
D_MODEL = 1024
MAX_LEN = 2048
EPS = 1e-5

NC, NS = 2, 16
NW = NC * NS

N_CHUNKS = 2
GATHER_CHUNK = 32
LN_ROWS = 512


def _pe_table(max_len, d_model):
    pos = np.arange(max_len, dtype=np.float32)[:, None]
    i = np.arange(0, d_model, 2, dtype=np.float32)
    div = np.exp(-np.log(10000.0) * i / d_model)
    pe = np.zeros((max_len, d_model), dtype=np.float32)
    pe[:, 0::2] = np.sin(pos * div)
    pe[:, 1::2] = np.cos(pos * div)
    return pe


_PE = _pe_table(MAX_LEN, D_MODEL)


def _sc_gather(table, idx):
    n = idx.shape[0]
    d = table.shape[1]
    b_per_w = n // NW
    chunk = min(GATHER_CHUNK, b_per_w)
    n_chunks = b_per_w // chunk
    mesh = plsc.VectorSubcoreMesh(core_axis_name="c", subcore_axis_name="s")

    @functools.partial(
        pl.kernel,
        out_type=jax.ShapeDtypeStruct((n, d), table.dtype),
        mesh=mesh,
        scratch_types=[
            pltpu.VMEM((b_per_w,), jnp.int32),
            pltpu.VMEM((chunk, d), table.dtype),
            pltpu.VMEM((chunk, d), table.dtype),
            pltpu.SemaphoreType.DMA,
            pltpu.SemaphoreType.DMA,
            pltpu.SemaphoreType.DMA,
            pltpu.SemaphoreType.DMA,
        ],
    )
    def gather_kernel(table_hbm, idx_hbm, out_hbm, idx_v, rows0, rows1,
                      gsem0, gsem1, wsem0, wsem1):
        wid = lax.axis_index("s") * NC + lax.axis_index("c")
        base = wid * b_per_w
        pltpu.sync_copy(idx_hbm.at[pl.ds(base, b_per_w)], idx_v)

        rows = (rows0, rows1)
        gsems = (gsem0, gsem1)
        wsems = (wsem0, wsem1)

        def start_gather(c, buf):
            return pltpu.async_copy(
                table_hbm.at[idx_v.at[pl.ds(c * chunk, chunk)]],
                rows[buf], gsems[buf])

        def start_write(c, buf):
            return pltpu.async_copy(
                rows[buf], out_hbm.at[pl.ds(base + c * chunk, chunk)],
                wsems[buf])

        start_gather(0, 0)
        writes = [None, None]
        for c in range(n_chunks):
            buf = c % 2
            pltpu.make_async_copy(
                table_hbm.at[idx_v.at[pl.ds(c * chunk, chunk)]],
                rows[buf], gsems[buf]).wait()
            if c + 1 < n_chunks:
                nbuf = (c + 1) % 2
                if writes[nbuf] is not None:
                    writes[nbuf].wait()
                    writes[nbuf] = None
                start_gather(c + 1, nbuf)
            writes[buf] = start_write(c, buf)
        for w in writes:
            if w is not None:
                w.wait()

    return gather_kernel(table, idx)


def _tc_ln_chunk(tok, pe, gamma, beta, out_prev, n_total, chunk_id):
    r, d = tok.shape
    l = pe.shape[0]
    blocks = r // LN_ROWS
    n_pe_blocks = l // LN_ROWS
    base = chunk_id * blocks

    def body(tok_ref, pe_ref, g_ref, b_ref, *rest):
        o_ref = rest[-1]
        i = pl.program_id(0)
        pe_blk = pe_ref[pl.ds((i % n_pe_blocks) * LN_ROWS, LN_ROWS), :]
        x = tok_ref[...] + pe_blk.astype(jnp.float32)
        m = jnp.mean(x, axis=-1, keepdims=True)
        xc = x - m
        v = jnp.mean(xc * xc, axis=-1, keepdims=True)
        o_ref[...] = xc * lax.rsqrt(v + EPS) * g_ref[...] + b_ref[...]

    in_specs = [
        pl.BlockSpec((LN_ROWS, d), lambda i: (i, 0)),
        pl.BlockSpec((l, d), lambda i: (0, 0)),
        pl.BlockSpec((1, d), lambda i: (0, 0)),
        pl.BlockSpec((1, d), lambda i: (0, 0)),
    ]
    args = [tok, pe, gamma.reshape(1, d), beta.reshape(1, d)]
    aliases = {}
    if out_prev is not None:
        in_specs.append(pl.BlockSpec(memory_space=pl.ANY))
        args.append(out_prev)
        aliases = {4: 0}

    return pl.pallas_call(
        body,
        grid=(blocks,),
        in_specs=in_specs,
        out_specs=pl.BlockSpec((LN_ROWS, d), lambda i: (base + i, 0)),
        out_shape=jax.ShapeDtypeStruct((n_total, d), jnp.float32),
        input_output_aliases=aliases,
    )(*args)


def kernel(sequence, table, gamma, beta):
    b, l = sequence.shape
    d = table.shape[1]
    n = b * l
    idx = sequence.reshape(-1).astype(jnp.int32)
    pe = jnp.asarray(_PE[:l], dtype=jnp.bfloat16)

    rows_per_chunk = n // N_CHUNKS
    toks = [
        _sc_gather(table, lax.dynamic_slice_in_dim(idx, c * rows_per_chunk,
                                                   rows_per_chunk))
        for c in range(N_CHUNKS)
    ]
    out = None
    for c in range(N_CHUNKS):
        out = _tc_ln_chunk(toks[c], pe, gamma, beta, out, n, c)
    return out.reshape(b, l, d)


# speedup vs baseline: 1.1895x; 1.0122x over previous
import functools

import jax
import jax.numpy as jnp
import numpy as np
from jax import lax
from jax.experimental import pallas as pl
from jax.experimental.pallas import tpu as pltpu
from jax.experimental.pallas import tpu_sc as plsc

VOCAB = 100000
D_MODEL = 1024
MAX_LEN = 2048
EPS = 1e-5

NC, NS = 2, 16
NW = NC * NS

N_CHUNKS = 2
GATHER_CHUNK = 32
LN_ROWS = 1024


def _pe_table(max_len, d_model):
    pos = np.arange(max_len, dtype=np.float32)[:, None]
    i = np.arange(0, d_model, 2, dtype=np.float32)
    div = np.exp(-np.log(10000.0) * i / d_model)
    pe = np.zeros((max_len, d_model), dtype=np.float32)
    pe[:, 0::2] = np.sin(pos * div)
    pe[:, 1::2] = np.cos(pos * div)
    return pe


_PE = _pe_table(MAX_LEN, D_MODEL)


def _sc_gather(table, idx):
    n = idx.shape[0]
    d = table.shape[1]
    b_per_w = n // NW
    chunk = min(GATHER_CHUNK, b_per_w)
    n_chunks = b_per_w // chunk
    mesh = plsc.VectorSubcoreMesh(core_axis_name="c", subcore_axis_name="s")

    @functools.partial(
        pl.kernel,
        out_type=jax.ShapeDtypeStruct((n, d), table.dtype),
        mesh=mesh,
        scratch_types=[
            pltpu.VMEM((b_per_w,), jnp.int32),
            pltpu.VMEM((chunk, d), table.dtype),
            pltpu.VMEM((chunk, d), table.dtype),
            pltpu.SemaphoreType.DMA,
            pltpu.SemaphoreType.DMA,
            pltpu.SemaphoreType.DMA,
            pltpu.SemaphoreType.DMA,
        ],
    )
    def gather_kernel(table_hbm, idx_hbm, out_hbm, idx_v, rows0, rows1,
                      gsem0, gsem1, wsem0, wsem1):
        wid = lax.axis_index("s") * NC + lax.axis_index("c")
        base = wid * b_per_w
        pltpu.sync_copy(idx_hbm.at[pl.ds(base, b_per_w)], idx_v)

        rows = (rows0, rows1)
        gsems = (gsem0, gsem1)
        wsems = (wsem0, wsem1)

        def start_gather(c, buf):
            return pltpu.async_copy(
                table_hbm.at[idx_v.at[pl.ds(c * chunk, chunk)]],
                rows[buf], gsems[buf])

        def start_write(c, buf):
            return pltpu.async_copy(
                rows[buf], out_hbm.at[pl.ds(base + c * chunk, chunk)],
                wsems[buf])

        start_gather(0, 0)
        writes = [None, None]
        for c in range(n_chunks):
            buf = c % 2
            pltpu.make_async_copy(
                table_hbm.at[idx_v.at[pl.ds(c * chunk, chunk)]],
                rows[buf], gsems[buf]).wait()
            if c + 1 < n_chunks:
                nbuf = (c + 1) % 2
                if writes[nbuf] is not None:
                    writes[nbuf].wait()
                    writes[nbuf] = None
                start_gather(c + 1, nbuf)
            writes[buf] = start_write(c, buf)
        for w in writes:
            if w is not None:
                w.wait()

    return gather_kernel(table, idx)


def _tc_ln_chunk(tok, pe, gamma, beta, out_prev, n_total, chunk_id):
    r, d = tok.shape
    l = pe.shape[0]
    blocks = r // LN_ROWS
    n_pe_blocks = l // LN_ROWS
    base = chunk_id * blocks

    def body(tok_ref, pe_ref, g_ref, b_ref, *rest):
        o_ref = rest[-1]
        i = pl.program_id(0)
        pe_blk = pe_ref[pl.ds((i % n_pe_blocks) * LN_ROWS, LN_ROWS), :]
        x = tok_ref[...] + pe_blk.astype(jnp.float32)
        m = jnp.mean(x, axis=-1, keepdims=True)
        xc = x - m
        v = jnp.mean(xc * xc, axis=-1, keepdims=True)
        o_ref[...] = xc * lax.rsqrt(v + EPS) * g_ref[...] + b_ref[...]

    in_specs = [
        pl.BlockSpec((LN_ROWS, d), lambda i: (i, 0)),
        pl.BlockSpec((l, d), lambda i: (0, 0)),
        pl.BlockSpec((1, d), lambda i: (0, 0)),
        pl.BlockSpec((1, d), lambda i: (0, 0)),
    ]
    args = [tok, pe, gamma.reshape(1, d), beta.reshape(1, d)]
    aliases = {}
    if out_prev is not None:
        in_specs.append(pl.BlockSpec(memory_space=pl.ANY))
        args.append(out_prev)
        aliases = {4: 0}

    return pl.pallas_call(
        body,
        grid=(blocks,),
        in_specs=in_specs,
        out_specs=pl.BlockSpec((LN_ROWS, d), lambda i: (base + i, 0)),
        out_shape=jax.ShapeDtypeStruct((n_total, d), jnp.float32),
        input_output_aliases=aliases,
    )(*args)


def kernel(sequence, table, gamma, beta):
    b, l = sequence.shape
    d = table.shape[1]
    n = b * l
    idx = sequence.reshape(-1).astype(jnp.int32)
    pe = jnp.asarray(_PE[:l], dtype=jnp.bfloat16)

    rows_per_chunk = n // N_CHUNKS
    toks = [
        _sc_gather(table, lax.dynamic_slice_in_dim(idx, c * rows_per_chunk,
                                                   rows_per_chunk))
        for c in range(N_CHUNKS)
    ]
    out = None
    for c in range(N_CHUNKS):
        out = _tc_ln_chunk(toks[c], pe, gamma, beta, out, n, c)
    return out.reshape(b, l, d)
